# Initial kernel scaffold; baseline (speedup 1.0000x reference)
import jax
import jax.numpy as jnp
from jax.experimental import pallas as pl


def kernel(inputs, table):
    raise NotImplementedError("write your pallas kernel here")


# speedup vs baseline: 1.8857x; 1.8857x over previous
import functools
import math

import jax
import jax.numpy as jnp
from jax import lax
from jax.experimental import pallas as pl
from jax.experimental.pallas import tpu as pltpu
from jax.experimental.pallas import tpu_sc as plsc

_CHUNK = 100


def _pe_body(out_ref):
    s, e = out_ref.shape
    pos = lax.broadcasted_iota(jnp.int32, (s, e), 0).astype(jnp.float32)
    ii = lax.broadcasted_iota(jnp.int32, (s, e), 1)
    i = ii.astype(jnp.float32)
    angle = pos * jnp.exp(i * (-2.0 / e) * math.log(10000.0))
    even = (ii % 2) == 0
    out_ref[...] = jnp.where(even, jnp.sin(angle), jnp.cos(angle))


def _make_pe(s, e):
    return pl.pallas_call(
        _pe_body,
        out_shape=jax.ShapeDtypeStruct((s, e), jnp.float32),
    )()


def _make_sc_kernel(nw, nc, chunks, s, e):
    mesh = plsc.VectorSubcoreMesh(core_axis_name="c", subcore_axis_name="s")

    @functools.partial(
        pl.kernel,
        mesh=mesh,
        compiler_params=pltpu.CompilerParams(use_tc_tiling_on_sc=False),
        out_type=jax.ShapeDtypeStruct((nw * chunks, _CHUNK, e), jnp.float32),
        scratch_types=[
            pltpu.VMEM((chunks, _CHUNK), jnp.int32),
            pltpu.VMEM((s, e), jnp.float32),
            pltpu.VMEM((_CHUNK, e), jnp.float32),
            pltpu.SemaphoreType.DMA,
        ],
    )
    def sc_kernel(idx_hbm, table_hbm, pe_hbm, out_hbm, idx_v, pe_v, rows_v, sem):
        wid = lax.axis_index("s") * nc + lax.axis_index("c")
        pltpu.sync_copy(idx_hbm.at[wid], idx_v)
        pltpu.sync_copy(pe_hbm, pe_v)

        def chunk_body(c, carry):
            pltpu.async_copy(table_hbm.at[idx_v.at[c]], rows_v, sem).wait()
            off = (c % 2) * _CHUNK

            def row_body(r, inner):
                for j in range(e // 16):
                    sl = pl.ds(16 * j, 16)
                    rows_v[r, sl] = rows_v[r, sl] + pe_v[off + r, sl]
                return inner

            lax.fori_loop(0, _CHUNK, row_body, 0)
            pltpu.sync_copy(rows_v, out_hbm.at[wid * chunks + c])
            return carry

        lax.fori_loop(0, chunks, chunk_body, 0)

    return sc_kernel


def kernel(inputs, table):
    b, s = inputs.shape
    v, e = table.shape
    info = plsc.get_sparse_core_info()
    nc, ns = info.num_cores, info.num_subcores
    nw = nc * ns
    total = b * s
    chunks = total // (nw * _CHUNK)

    pe = _make_pe(s, e)
    idx = inputs.astype(jnp.int32).reshape(nw, chunks, _CHUNK)
    out = _make_sc_kernel(nw, nc, chunks, s, e)(idx, table, pe)
    return out.reshape(b, s, e)


# speedup vs baseline: 3.0493x; 1.6170x over previous
import functools
import math

import jax
import jax.numpy as jnp
from jax import lax
from jax.experimental import pallas as pl
from jax.experimental.pallas import tpu as pltpu
from jax.experimental.pallas import tpu_sc as plsc

_CHUNK = 100


def _pe_body(out_ref):
    s, e = out_ref.shape
    pos = lax.broadcasted_iota(jnp.int32, (s, e), 0).astype(jnp.float32)
    ii = lax.broadcasted_iota(jnp.int32, (s, e), 1)
    i = ii.astype(jnp.float32)
    angle = pos * jnp.exp(i * (-2.0 / e) * math.log(10000.0))
    even = (ii % 2) == 0
    out_ref[...] = jnp.where(even, jnp.sin(angle), jnp.cos(angle))


def _make_pe(s, e):
    return pl.pallas_call(
        _pe_body,
        out_shape=jax.ShapeDtypeStruct((s, e), jnp.float32),
    )()


def _make_sc_kernel(nw, nc, chunks, s, e):
    mesh = plsc.VectorSubcoreMesh(core_axis_name="c", subcore_axis_name="s")

    @functools.partial(
        pl.kernel,
        mesh=mesh,
        compiler_params=pltpu.CompilerParams(use_tc_tiling_on_sc=False),
        out_type=jax.ShapeDtypeStruct((nw * chunks, _CHUNK, e), jnp.float32),
        scratch_types=[
            pltpu.VMEM((chunks, _CHUNK), jnp.int32),
            pltpu.VMEM((s, e), jnp.float32),
            pltpu.VMEM((_CHUNK, e), jnp.float32),
            pltpu.VMEM((_CHUNK, e), jnp.float32),
            pltpu.VMEM((_CHUNK, e), jnp.float32),
            pltpu.VMEM((_CHUNK, e), jnp.float32),
            pltpu.SemaphoreType.DMA,
            pltpu.SemaphoreType.DMA,
            pltpu.SemaphoreType.DMA,
            pltpu.SemaphoreType.DMA,
        ],
    )
    def sc_kernel(idx_hbm, table_hbm, pe_hbm, out_hbm, idx_v, pe_v,
                  rows0, rows1, st0, st1, gsem0, gsem1, wsem0, wsem1):
        rows = (rows0, rows1)
        st = (st0, st1)
        gsem = (gsem0, gsem1)
        wsem = (wsem0, wsem1)

        wid = lax.axis_index("s") * nc + lax.axis_index("c")
        w0 = wid * chunks
        pltpu.sync_copy(idx_hbm.at[wid], idx_v)
        pltpu.sync_copy(pe_hbm, pe_v)

        def gather_start(c, b):
            pltpu.async_copy(table_hbm.at[idx_v.at[c]], rows[b], gsem[b])

        def gather_wait(c, b):
            pltpu.make_async_copy(
                table_hbm.at[idx_v.at[c]], rows[b], gsem[b]).wait()

        def write_start(c, b):
            pltpu.async_copy(st[b], out_hbm.at[w0 + c], wsem[b])

        def write_wait(c, b):
            pltpu.make_async_copy(st[b], out_hbm.at[w0 + c], wsem[b]).wait()

        def add_pe(b):
            pe_off = (b % 2) * _CHUNK

            @plsc.parallel_loop(0, _CHUNK, unroll=10)
            def _(r):
                for j in range(e // 16):
                    sl = pl.ds(16 * j, 16)
                    st[b][r, sl] = rows[b][r, sl] + pe_v[pe_off + r, sl]

        gather_start(0, 0)
        gather_start(1, 1)

        for c in (0, 1):
            b = c
            gather_wait(c, b)
            add_pe(b)
            gather_start(c + 2, b)
            write_start(c, b)

        @pl.loop(2, chunks - 2, step=2)
        def _(g):
            for b in range(2):
                c = g + b
                gather_wait(c, b)
                write_wait(c - 2, b)
                add_pe(b)
                gather_start(c + 2, b)
                write_start(c, b)

        for c in (chunks - 2, chunks - 1):
            b = c % 2
            gather_wait(c, b)
            write_wait(c - 2, b)
            add_pe(b)
            write_start(c, b)

        write_wait(chunks - 2, (chunks - 2) % 2)
        write_wait(chunks - 1, (chunks - 1) % 2)

    return sc_kernel


def kernel(inputs, table):
    b, s = inputs.shape
    v, e = table.shape
    info = plsc.get_sparse_core_info()
    nc, ns = info.num_cores, info.num_subcores
    nw = nc * ns
    total = b * s
    chunks = total // (nw * _CHUNK)

    pe = _make_pe(s, e)
    idx = inputs.astype(jnp.int32).reshape(nw, chunks, _CHUNK)
    out = _make_sc_kernel(nw, nc, chunks, s, e)(idx, table, pe)
    return out.reshape(b, s, e)


# speedup vs baseline: 3.3469x; 1.0976x over previous
import functools
import math

import jax
import jax.numpy as jnp
from jax import lax
from jax.experimental import pallas as pl
from jax.experimental.pallas import tpu as pltpu
from jax.experimental.pallas import tpu_sc as plsc

_P = 4
_L = 16


def _pe_body(out_ref):
    e, s, l = out_ref.shape
    ch = lax.broadcasted_iota(jnp.int32, (e, s, l), 0)
    pos = lax.broadcasted_iota(jnp.int32, (e, s, l), 1).astype(jnp.float32)
    angle = pos * jnp.exp(ch.astype(jnp.float32) * (-2.0 / e) * math.log(10000.0))
    even = (ch % 2) == 0
    out_ref[...] = jnp.where(even, jnp.sin(angle), jnp.cos(angle))


def _make_pe(s, e):
    return pl.pallas_call(
        _pe_body,
        out_shape=jax.ShapeDtypeStruct((e, s, _L), jnp.float32),
    )()


def _make_sc_kernel(nw, nc, b, s, e, v):
    mesh = plsc.VectorSubcoreMesh(core_axis_name="c", subcore_axis_name="s")
    n_chunks = s // _P
    ch_per_tile = e // nw

    @functools.partial(
        pl.kernel,
        mesh=mesh,
        compiler_params=pltpu.CompilerParams(
            use_tc_tiling_on_sc=False, needs_layout_passes=False),
        out_type=jax.ShapeDtypeStruct((s, e, b), jnp.float32),
        scratch_types=[
            pltpu.VMEM((v,), jnp.float32),
            pltpu.VMEM((s, _L), jnp.float32),
            pltpu.VMEM((_P, b), jnp.int32),
            pltpu.VMEM((_P, b), jnp.int32),
            pltpu.VMEM((_P, b), jnp.float32),
            pltpu.VMEM((_P, b), jnp.float32),
            pltpu.SemaphoreType.DMA,
            pltpu.SemaphoreType.DMA,
            pltpu.SemaphoreType.DMA,
            pltpu.SemaphoreType.DMA,
        ],
    )
    def sc_kernel(tbl_hbm, idx_hbm, pe_hbm, out_hbm, col_v, pe_v,
                  idx0, idx1, out0, out1, isem0, isem1, wsem0, wsem1):
        idx_v = (idx0, idx1)
        out_v = (out0, out1)
        isem = (isem0, isem1)
        wsem = (wsem0, wsem1)

        tid = lax.axis_index("s") * nc + lax.axis_index("c")

        def idx_start(c, bi):
            pltpu.async_copy(idx_hbm.at[pl.ds(c * _P, _P)], idx_v[bi], isem[bi])

        def idx_wait(c, bi):
            pltpu.make_async_copy(
                idx_hbm.at[pl.ds(c * _P, _P)], idx_v[bi], isem[bi]).wait()

        def write_start(c, bi, ch):
            pltpu.async_copy(
                out_v[bi], out_hbm.at[pl.ds(c * _P, _P), ch], wsem[bi])

        def write_wait(c, bi, ch):
            pltpu.make_async_copy(
                out_v[bi], out_hbm.at[pl.ds(c * _P, _P), ch], wsem[bi]).wait()

        def process(c, bi):
            for sp in range(_P):
                pe16 = pe_v[c * _P + sp, :]

                @plsc.parallel_loop(0, b, step=_L, unroll=8)
                def _(i):
                    iv = idx_v[bi][sp, pl.ds(i, _L)]
                    vals = plsc.load_gather(col_v, [iv])
                    out_v[bi][sp, pl.ds(i, _L)] = vals + pe16

        for cpass in range(ch_per_tile):
            ch = tid * ch_per_tile + cpass
            pltpu.sync_copy(tbl_hbm.at[ch], col_v)
            pltpu.sync_copy(pe_hbm.at[ch], pe_v)

            idx_start(0, 0)
            idx_start(1, 1)

            for c in (0, 1):
                bi = c
                idx_wait(c, bi)
                process(c, bi)
                idx_start(c + 2, bi)
                write_start(c, bi, ch)

            @pl.loop(2, n_chunks - 2, step=2)
            def _(g):
                for bi in range(2):
                    c = g + bi
                    idx_wait(c, bi)
                    write_wait(c - 2, bi, ch)
                    process(c, bi)
                    idx_start(c + 2, bi)
                    write_start(c, bi, ch)

            for c in (n_chunks - 2, n_chunks - 1):
                bi = c % 2
                idx_wait(c, bi)
                write_wait(c - 2, bi, ch)
                process(c, bi)
                write_start(c, bi, ch)

            write_wait(n_chunks - 2, (n_chunks - 2) % 2, ch)
            write_wait(n_chunks - 1, (n_chunks - 1) % 2, ch)

    return sc_kernel


def kernel(inputs, table):
    b, s = inputs.shape
    v, e = table.shape
    info = plsc.get_sparse_core_info()
    nc, ns = info.num_cores, info.num_subcores
    nw = nc * ns

    tbl_t = table.T
    idx_t = inputs.astype(jnp.int32).T
    pe = _make_pe(s, e)
    out_t = _make_sc_kernel(nw, nc, b, s, e, v)(tbl_t, idx_t, pe)
    return out_t.transpose(2, 0, 1)


# speedup vs baseline: 3.6144x; 1.0799x over previous
import functools
import math

import jax
import jax.numpy as jnp
import numpy as np
from jax import lax
from jax.experimental import pallas as pl
from jax.experimental.pallas import tpu as pltpu
from jax.experimental.pallas import tpu_sc as plsc

_P = 4
_L = 16


def _make_pe(s, e):
    ch = np.arange(e, dtype=np.float64)[:, None]
    pos = np.arange(s, dtype=np.float64)[None, :]
    angle = pos * np.power(10000.0, -2.0 * ch / float(e))
    pe = np.where((np.arange(e) % 2 == 0)[:, None], np.sin(angle), np.cos(angle))
    pe = np.broadcast_to(pe.astype(np.float32)[:, :, None], (e, s, _L))
    return jnp.asarray(pe)


def _make_sc_kernel(nw, nc, b, s, e, v):
    mesh = plsc.VectorSubcoreMesh(core_axis_name="c", subcore_axis_name="s")
    n_chunks = s // _P
    ch_per_tile = e // nw

    @functools.partial(
        pl.kernel,
        mesh=mesh,
        compiler_params=pltpu.CompilerParams(
            use_tc_tiling_on_sc=False, needs_layout_passes=False),
        out_type=jax.ShapeDtypeStruct((s, e, b), jnp.float32),
        scratch_types=[
            pltpu.VMEM((v,), jnp.float32),
            pltpu.VMEM((s, _L), jnp.float32),
            pltpu.VMEM((_P, b), jnp.int32),
            pltpu.VMEM((_P, b), jnp.int32),
            pltpu.VMEM((_P, b), jnp.float32),
            pltpu.VMEM((_P, b), jnp.float32),
            pltpu.SemaphoreType.DMA,
            pltpu.SemaphoreType.DMA,
            pltpu.SemaphoreType.DMA,
            pltpu.SemaphoreType.DMA,
        ],
    )
    def sc_kernel(tbl_hbm, idx_hbm, pe_hbm, out_hbm, col_v, pe_v,
                  idx0, idx1, out0, out1, isem0, isem1, wsem0, wsem1):
        idx_v = (idx0, idx1)
        out_v = (out0, out1)
        isem = (isem0, isem1)
        wsem = (wsem0, wsem1)

        tid = lax.axis_index("s") * nc + lax.axis_index("c")

        def idx_start(c, bi):
            pltpu.async_copy(idx_hbm.at[pl.ds(c * _P, _P)], idx_v[bi], isem[bi])

        def idx_wait(c, bi):
            pltpu.make_async_copy(
                idx_hbm.at[pl.ds(c * _P, _P)], idx_v[bi], isem[bi]).wait()

        def write_start(c, bi, ch):
            pltpu.async_copy(
                out_v[bi], out_hbm.at[pl.ds(c * _P, _P), ch], wsem[bi])

        def write_wait(c, bi, ch):
            pltpu.make_async_copy(
                out_v[bi], out_hbm.at[pl.ds(c * _P, _P), ch], wsem[bi]).wait()

        def process(c, bi):
            for sp in range(_P):
                pe16 = pe_v[c * _P + sp, :]

                @plsc.parallel_loop(0, b, step=_L, unroll=8)
                def _(i):
                    iv = idx_v[bi][sp, pl.ds(i, _L)]
                    vals = plsc.load_gather(col_v, [iv])
                    out_v[bi][sp, pl.ds(i, _L)] = vals + pe16

        for cpass in range(ch_per_tile):
            ch = tid * ch_per_tile + cpass
            pltpu.sync_copy(tbl_hbm.at[ch], col_v)
            pltpu.sync_copy(pe_hbm.at[ch], pe_v)

            idx_start(0, 0)
            idx_start(1, 1)

            for c in (0, 1):
                bi = c
                idx_wait(c, bi)
                process(c, bi)
                idx_start(c + 2, bi)
                write_start(c, bi, ch)

            @pl.loop(2, n_chunks - 2, step=2)
            def _(g):
                for bi in range(2):
                    c = g + bi
                    idx_wait(c, bi)
                    write_wait(c - 2, bi, ch)
                    process(c, bi)
                    idx_start(c + 2, bi)
                    write_start(c, bi, ch)

            for c in (n_chunks - 2, n_chunks - 1):
                bi = c % 2
                idx_wait(c, bi)
                write_wait(c - 2, bi, ch)
                process(c, bi)
                write_start(c, bi, ch)

            write_wait(n_chunks - 2, (n_chunks - 2) % 2, ch)
            write_wait(n_chunks - 1, (n_chunks - 1) % 2, ch)

    return sc_kernel


def kernel(inputs, table):
    b, s = inputs.shape
    v, e = table.shape
    info = plsc.get_sparse_core_info()
    nc, ns = info.num_cores, info.num_subcores
    nw = nc * ns

    tbl_t = table.T
    idx_t = inputs.astype(jnp.int32).T
    pe = _make_pe(s, e)
    out_t = _make_sc_kernel(nw, nc, b, s, e, v)(tbl_t, idx_t, pe)
    return out_t.transpose(2, 0, 1)


# speedup vs baseline: 4.7795x; 1.3224x over previous
import functools
import math

import jax
import jax.numpy as jnp
import numpy as np
from jax import lax
from jax.experimental import pallas as pl
from jax.experimental.pallas import tpu as pltpu
from jax.experimental.pallas import tpu_sc as plsc

_P = 4
_L = 16


def _make_pe(s, e):
    ch = np.arange(e, dtype=np.float64)[:, None]
    pos = np.arange(s, dtype=np.float64)[None, :]
    angle = pos * np.power(10000.0, -2.0 * ch / float(e))
    pe = np.where((np.arange(e) % 2 == 0)[:, None], np.sin(angle), np.cos(angle))
    pe = np.broadcast_to(pe.astype(np.float32)[:, :, None], (e, s, _L))
    return jnp.asarray(pe)


def _make_sc_kernel(nw, nc, b, s, e, v):
    mesh = plsc.VectorSubcoreMesh(core_axis_name="c", subcore_axis_name="s")
    n_chunks = s // _P
    ch_per_tile = e // nw

    @functools.partial(
        pl.kernel,
        mesh=mesh,
        compiler_params=pltpu.CompilerParams(
            use_tc_tiling_on_sc=False, needs_layout_passes=False),
        out_type=jax.ShapeDtypeStruct((s, e // 8, b // 128, 8, 128), jnp.float32),
        scratch_types=[
            pltpu.VMEM((v,), jnp.float32),
            pltpu.VMEM((s, _L), jnp.float32),
            pltpu.VMEM((_P, b), jnp.int32),
            pltpu.VMEM((_P, b), jnp.int32),
            pltpu.VMEM((_P, b // 128, 128), jnp.float32),
            pltpu.VMEM((_P, b // 128, 128), jnp.float32),
            pltpu.SemaphoreType.DMA,
            pltpu.SemaphoreType.DMA,
            pltpu.SemaphoreType.DMA,
            pltpu.SemaphoreType.DMA,
        ],
    )
    def sc_kernel(tbl_hbm, idx_hbm, pe_hbm, out_hbm, col_v, pe_v,
                  idx0, idx1, out0, out1, isem0, isem1, wsem0, wsem1):
        idx_v = (idx0, idx1)
        out_v = (out0, out1)
        isem = (isem0, isem1)
        wsem = (wsem0, wsem1)

        tid = lax.axis_index("s") * nc + lax.axis_index("c")

        def idx_start(c, bi):
            pltpu.async_copy(idx_hbm.at[pl.ds(c * _P, _P)], idx_v[bi], isem[bi])

        def idx_wait(c, bi):
            pltpu.make_async_copy(
                idx_hbm.at[pl.ds(c * _P, _P)], idx_v[bi], isem[bi]).wait()

        def write_start(c, bi, tr, r):
            pltpu.async_copy(
                out_v[bi], out_hbm.at[pl.ds(c * _P, _P), tr, :, r, :], wsem[bi])

        def write_wait(c, bi, tr, r):
            pltpu.make_async_copy(
                out_v[bi], out_hbm.at[pl.ds(c * _P, _P), tr, :, r, :],
                wsem[bi]).wait()

        def process(c, bi):
            for sp in range(_P):
                pe16 = pe_v[c * _P + sp, :]

                @plsc.parallel_loop(0, b, step=_L, unroll=8)
                def _(i):
                    iv = idx_v[bi][sp, pl.ds(i, _L)]
                    vals = plsc.load_gather(col_v, [iv])
                    out_v[bi][sp, i // 128, pl.ds(i % 128, _L)] = vals + pe16

        for cpass in range(ch_per_tile):
            ch = tid * ch_per_tile + cpass
            tr = ch // 8
            r = ch % 8
            pltpu.sync_copy(tbl_hbm.at[ch], col_v)
            pltpu.sync_copy(pe_hbm.at[ch], pe_v)

            idx_start(0, 0)
            idx_start(1, 1)

            for c in (0, 1):
                bi = c
                idx_wait(c, bi)
                process(c, bi)
                idx_start(c + 2, bi)
                write_start(c, bi, tr, r)

            @pl.loop(2, n_chunks - 2, step=2)
            def _(g):
                for bi in range(2):
                    c = g + bi
                    idx_wait(c, bi)
                    write_wait(c - 2, bi, tr, r)
                    process(c, bi)
                    idx_start(c + 2, bi)
                    write_start(c, bi, tr, r)

            for c in (n_chunks - 2, n_chunks - 1):
                bi = c % 2
                idx_wait(c, bi)
                write_wait(c - 2, bi, tr, r)
                process(c, bi)
                write_start(c, bi, tr, r)

            write_wait(n_chunks - 2, (n_chunks - 2) % 2, tr, r)
            write_wait(n_chunks - 1, (n_chunks - 1) % 2, tr, r)

    return sc_kernel


def kernel(inputs, table):
    b, s = inputs.shape
    v, e = table.shape
    info = plsc.get_sparse_core_info()
    nc, ns = info.num_cores, info.num_subcores
    nw = nc * ns

    tbl_t = table.T
    idx_t = inputs.astype(jnp.int32).T
    pe = _make_pe(s, e)
    out2 = _make_sc_kernel(nw, nc, b, s, e, v)(tbl_t, idx_t, pe)
    out_t = out2.transpose(0, 1, 3, 2, 4).reshape(s, e, b)
    return out_t.transpose(2, 0, 1)
